# baseline (speedup 1.0000x reference)
import functools
import jax
import jax.numpy as jnp
from jax import lax
from jax.experimental import pallas as pl
from jax.experimental.pallas import tpu as pltpu
from jax.experimental.pallas import tpu_sc as plsc

N_K = 50000
H_K = 64
R_K = 4
L_K = 3
E_K = 800000
E_PAD = 802816
PAD_DST = 50001

N_HALF = 25000
TRASH_ROW = 25000
ACC_ROWS = 25008
CH = 112
CPB = 4
EB = CH * H_K * 4

CT_HALF = 102400
CT_TRASH = 100096
INV_SZ = 2 * CT_HALF

EPT_P1 = 50176
EPW_P2 = 25088

_MESH = plsc.VectorSubcoreMesh(core_axis_name="c", subcore_axis_name="s")
_SC_PARAMS = pltpu.CompilerParams(needs_layout_passes=False,
                                  use_tc_tiling_on_sc=False)


def _user_mm_body(x_ref, w_ref, b_ref, o_ref):
    o_ref[...] = jnp.dot(x_ref[...], w_ref[...],
                         preferred_element_type=jnp.float32) + b_ref[...]


def _user_matmul(x_user, W_user, b_user):
    M, K = x_user.shape
    H = W_user.shape[1]
    BM = 1000
    return pl.pallas_call(
        _user_mm_body,
        grid=(M // BM,),
        in_specs=[
            pl.BlockSpec((BM, K), lambda i: (i, 0)),
            pl.BlockSpec((K, H), lambda i: (0, 0)),
            pl.BlockSpec((1, H), lambda i: (0, 0)),
        ],
        out_specs=pl.BlockSpec((BM, H), lambda i: (i, 0)),
        out_shape=jax.ShapeDtypeStruct((M, H), jnp.float32),
    )(x_user, W_user, b_user.reshape(1, H))


def _layer_mm_body(relu, x_ref, wroot_ref, wrel_ref, bias_ref, root_ref, y_ref):
    x = x_ref[...]
    if relu:
        x = jnp.maximum(x, 0.0)
    root_ref[...] = jnp.dot(x, wroot_ref[...],
                            preferred_element_type=jnp.float32) + bias_ref[...]
    for r in range(R_K):
        y_ref[r] = jnp.dot(x, wrel_ref[r],
                           preferred_element_type=jnp.float32)


def _layer_matmul(x, W_root_l, W_rel_l, bias_l, relu):
    N, H = x.shape
    BM = 1000
    return pl.pallas_call(
        functools.partial(_layer_mm_body, relu),
        grid=(N // BM,),
        in_specs=[
            pl.BlockSpec((BM, H), lambda i: (i, 0)),
            pl.BlockSpec((H, H), lambda i: (0, 0)),
            pl.BlockSpec((R_K, H, H), lambda i: (0, 0, 0)),
            pl.BlockSpec((1, H), lambda i: (0, 0)),
        ],
        out_specs=[
            pl.BlockSpec((BM, H), lambda i: (i, 0)),
            pl.BlockSpec((R_K, BM, H), lambda i: (0, i, 0)),
        ],
        out_shape=[
            jax.ShapeDtypeStruct((N, H), jnp.float32),
            jax.ShapeDtypeStruct((R_K, N, H), jnp.float32),
        ],
    )(x, W_root_l, W_rel_l, bias_l.reshape(1, H))


@functools.partial(
    pl.kernel, mesh=_MESH, compiler_params=_SC_PARAMS,
    out_type=jax.ShapeDtypeStruct((INV_SZ,), jnp.float32),
    scratch_types=[
        pltpu.VMEM_SHARED((CT_HALF,), jnp.float32),
        pltpu.VMEM((6400,), jnp.float32),
        pltpu.VMEM((512,), jnp.float32),
        pltpu.VMEM((512,), jnp.int32),
        pltpu.VMEM((512,), jnp.int32),
        pltpu.VMEM((512,), jnp.int32),
        pltpu.VMEM((512,), jnp.int32),
        pltpu.VMEM((512,), jnp.int32),
        pltpu.VMEM((512,), jnp.int32),
        pltpu.SemaphoreType.DMA,
        pltpu.SemaphoreType.DMA,
        pltpu.SemaphoreType.DMA,
        pltpu.SemaphoreType.DMA,
    ],
)
def _sc_prep1(dst_hbm, t_hbm, inv_hbm, counts_sh, zbuf, ones_b,
              db0, db1, tb0, tb1, cb0, cb1, sem_i0, sem_i1, sem_s0, sem_s1):
    cid = lax.axis_index("c")
    sid = lax.axis_index("s")
    dref = [db0, db1]
    tref = [tb0, tb1]
    cref = [cb0, cb1]
    sem_i = [sem_i0, sem_i1]
    NCH = EPT_P1 // 512

    def zero_body(i, _):
        zbuf[pl.ds(i * 16, 16)] = jnp.zeros((16,), jnp.float32)
        return 0
    lax.fori_loop(0, 400, zero_body, 0)

    def ones_body(i, _):
        ones_b[pl.ds(i * 16, 16)] = jnp.ones((16,), jnp.float32)
        return 0
    lax.fori_loop(0, 32, ones_body, 0)

    pltpu.sync_copy(zbuf.at[pl.ds(0, 6400)],
                    counts_sh.at[pl.ds(sid * 6400, 6400)])
    plsc.subcore_barrier()

    nbase = cid * N_HALF
    ebase = sid * EPT_P1

    def load(ch, w):
        pltpu.async_copy(dst_hbm.at[pl.ds(ebase + ch * 512, 512)],
                         dref[w], sem_i[w])
        pltpu.async_copy(t_hbm.at[pl.ds(ebase + ch * 512, 512)],
                         tref[w], sem_i[w])

    def wait_load(w):
        pltpu.make_async_copy(dst_hbm.at[pl.ds(0, 512)], dref[w],
                              sem_i[w]).wait()
        pltpu.make_async_copy(t_hbm.at[pl.ds(0, 512)], tref[w],
                              sem_i[w]).wait()

    sem_s = [sem_s0, sem_s1]

    def wait_scat(w):
        pltpu.make_async_copy(ones_b, counts_sh.at[pl.ds(0, 512)],
                              sem_s[w]).wait()

    def process(ch, w, first):
        wait_load(w)
        if not first:
            wait_scat(w)
        cb = cref[w]

        def cbody(k, _c):
            sl = pl.ds(k * 16, 16)
            d16 = dref[w][sl]
            t16 = tref[w][sl]
            loc = d16 - nbase
            ok = (loc >= 0) & (loc < N_HALF)
            cb[sl] = jnp.where(ok, loc * R_K + t16, CT_TRASH)
            return 0
        lax.fori_loop(0, 32, cbody, 0)
        pltpu.async_copy(ones_b, counts_sh.at[cb], sem_s[w], add=True)

    load(0, 0)
    load(1, 1)
    process(0, 0, True)
    load(2, 0)
    process(1, 1, True)
    load(3, 1)
    process(2, 0, False)

    def ch_loop(m, _):
        ch = 2 * m + 3
        load(ch + 1, 0)
        process(ch, 1, False)

        @pl.when(ch + 2 < NCH)
        def _l1():
            load(ch + 2, 1)
        process(ch + 1, 0, False)
        return 0
    lax.fori_loop(0, (NCH - 4) // 2, ch_loop, 0)

    process(NCH - 1, 1, False)
    wait_scat(0)
    wait_scat(1)

    plsc.subcore_barrier()
    pltpu.sync_copy(counts_sh.at[pl.ds(sid * 6400, 6400)],
                    zbuf.at[pl.ds(0, 6400)])

    def inv_body(i, _):
        v = zbuf[pl.ds(i * 16, 16)]
        zbuf[pl.ds(i * 16, 16)] = 1.0 / jnp.maximum(v, 1.0)
        return 0
    lax.fori_loop(0, 400, inv_body, 0)
    pltpu.sync_copy(zbuf.at[pl.ds(0, 6400)],
                    inv_hbm.at[pl.ds(cid * CT_HALF + sid * 6400, 6400)])


@functools.partial(
    pl.kernel, mesh=_MESH, compiler_params=_SC_PARAMS,
    out_type=[
        jax.ShapeDtypeStruct((E_PAD,), jnp.int32),
        jax.ShapeDtypeStruct((E_PAD,), jnp.float32),
        jax.ShapeDtypeStruct((2 * E_PAD,), jnp.int32),
    ],
    scratch_types=[
        pltpu.VMEM((512,), jnp.int32),
        pltpu.VMEM((512,), jnp.int32),
        pltpu.VMEM((512,), jnp.int32),
        pltpu.VMEM((512,), jnp.int32),
        pltpu.VMEM((512,), jnp.int32),
        pltpu.VMEM((512,), jnp.int32),
        pltpu.VMEM((512,), jnp.int32),
        pltpu.VMEM((512,), jnp.int32),
        pltpu.VMEM((512,), jnp.int32),
        pltpu.VMEM((512,), jnp.int32),
        pltpu.VMEM((512,), jnp.float32),
        pltpu.VMEM((512,), jnp.float32),
        pltpu.VMEM((512,), jnp.int32),
        pltpu.VMEM((512,), jnp.int32),
        pltpu.VMEM((512,), jnp.int32),
        pltpu.VMEM((512,), jnp.int32),
        pltpu.SemaphoreType.DMA,
        pltpu.SemaphoreType.DMA,
        pltpu.SemaphoreType.DMA,
        pltpu.SemaphoreType.DMA,
        pltpu.SemaphoreType.DMA,
    ],
)
def _sc_prep2(src_hbm, dst_hbm, t_hbm, inv_hbm, gidx_hbm, s_hbm, dl_hbm,
              bsrc0, bsrc1, bdst0, bdst1, bt0, bt1, gf0, gf1, cf0, cf1,
              sf0, sf1, d0f0, d0f1, d1f0, d1f1,
              sem_l0, sem_l1, sem_g, sem_st0, sem_st1):
    cid = lax.axis_index("c")
    sid = lax.axis_index("s")
    wid = cid * 16 + sid
    wbase = wid * EPW_P2
    NCW = EPW_P2 // 512

    bsrc = [bsrc0, bsrc1]
    bdst = [bdst0, bdst1]
    bt = [bt0, bt1]
    gf = [gf0, gf1]
    cf = [cf0, cf1]
    sf = [sf0, sf1]
    d0f = [d0f0, d0f1]
    d1f = [d1f0, d1f1]
    sem_st = [sem_st0, sem_st1]
    sem_l = [sem_l0, sem_l1]

    def load(ch, w):
        base = wbase + ch * 512
        pltpu.async_copy(src_hbm.at[pl.ds(base, 512)], bsrc[w], sem_l[w])
        pltpu.async_copy(dst_hbm.at[pl.ds(base, 512)], bdst[w], sem_l[w])
        pltpu.async_copy(t_hbm.at[pl.ds(base, 512)], bt[w], sem_l[w])

    def wait_load(w):
        for _ in range(3):
            pltpu.make_async_copy(src_hbm.at[pl.ds(0, 512)], bsrc[w],
                                  sem_l[w]).wait()

    def wait_st(w):
        for _ in range(3):
            pltpu.make_async_copy(gf[w], gidx_hbm.at[pl.ds(0, 512)],
                                  sem_st[w]).wait()
        pltpu.make_async_copy(sf[w], s_hbm.at[pl.ds(0, 512)],
                              sem_st[w]).wait()

    def process(ch, w, first):
        base = wbase + ch * 512
        wait_load(w)
        if not first:
            wait_st(w)

        def cbody(i, _c):
            sl = pl.ds(i * 16, 16)
            s16 = bsrc[w][sl]
            d16 = bdst[w][sl]
            t16 = bt[w][sl]
            gf[w][sl] = t16 * N_K + s16
            upper = d16 >= N_HALF
            loc = d16 - jnp.where(upper, N_HALF, 0)
            cf[w][sl] = jnp.where(upper, CT_HALF, 0) + loc * R_K + t16
            d0f[w][sl] = jnp.where(d16 < N_HALF, d16, TRASH_ROW)
            loc1 = d16 - N_HALF
            ok1 = (loc1 >= 0) & (loc1 < N_HALF)
            d1f[w][sl] = jnp.where(ok1, loc1, TRASH_ROW)
            return 0
        lax.fori_loop(0, 32, cbody, 0)

        @pl.when(ch + 2 < NCW)
        def _prefetch():
            load(ch + 2, w)
        pltpu.async_copy(inv_hbm.at[cf[w]], sf[w], sem_g)
        pltpu.async_copy(gf[w], gidx_hbm.at[pl.ds(base, 512)], sem_st[w])
        pltpu.async_copy(d0f[w], dl_hbm.at[pl.ds(base, 512)], sem_st[w])
        pltpu.async_copy(d1f[w], dl_hbm.at[pl.ds(E_PAD + base, 512)],
                         sem_st[w])
        pltpu.make_async_copy(inv_hbm.at[pl.ds(0, 512)], sf[w], sem_g).wait()
        pltpu.async_copy(sf[w], s_hbm.at[pl.ds(base, 512)], sem_st[w])

    load(0, 0)
    load(1, 1)
    process(0, 0, True)
    process(1, 1, True)

    def ch_loop(m, _):
        process(2 * m + 2, 0, False)
        process(2 * m + 3, 1, False)
        return 0
    lax.fori_loop(0, (NCW - 3) // 2, ch_loop, 0)

    process(NCW - 1, 0, False)
    wait_st(1)
    wait_st(0)


@functools.partial(
    pl.kernel, mesh=_MESH, compiler_params=_SC_PARAMS,
    out_type=jax.ShapeDtypeStruct((N_K, H_K), jnp.float32),
    scratch_types=[
        pltpu.VMEM_SHARED((ACC_ROWS, H_K), jnp.float32),
        pltpu.VMEM((CPB * CH,), jnp.int32),
        pltpu.VMEM((CPB * CH,), jnp.int32),
        pltpu.VMEM((CPB * CH,), jnp.float32),
        pltpu.VMEM((CPB * CH,), jnp.float32),
        pltpu.VMEM((CPB * CH,), jnp.int32),
        pltpu.VMEM((CPB * CH,), jnp.int32),
        pltpu.VMEM((CH, H_K), jnp.float32),
        pltpu.VMEM((CH, H_K), jnp.float32),
        pltpu.SemaphoreType.DMA,
        pltpu.SemaphoreType.DMA,
        pltpu.SemaphoreType.DMA,
        pltpu.SemaphoreType.DMA,
        pltpu.SemaphoreType.DMA,
    ],
)
def _sc_layer(root_hbm, y_hbm, gidx_hbm, s_hbm, dl_hbm, out_hbm,
              acc_sh, gsup0, gsup1, ssup0, ssup1, dsup0, dsup1,
              rows0, rows1, sem_g0, sem_g1, sem_s, sem_i0, sem_i1):
    cid = lax.axis_index("c")
    sid = lax.axis_index("s")
    nbase = cid * N_HALF
    r0 = sid * 1568
    SUP = CPB * CH
    NSUP = EPT_P1 // SUP

    @pl.when(sid < 15)
    def _init_main():
        pltpu.sync_copy(root_hbm.at[pl.ds(nbase + r0, 1568)],
                        acc_sh.at[pl.ds(r0, 1568)])

    @pl.when(sid == 15)
    def _init_tail():
        pltpu.sync_copy(root_hbm.at[pl.ds(nbase + 23520, 1480)],
                        acc_sh.at[pl.ds(23520, 1480)])

    plsc.subcore_barrier()

    ebase = sid * EPT_P1
    dlbase = cid * E_PAD + ebase
    gref = [gsup0, gsup1]
    sref = [ssup0, ssup1]
    dref = [dsup0, dsup1]
    sem_i = [sem_i0, sem_i1]
    sem_g = [sem_g0, sem_g1]

    def load_idx(b, w):
        eb = ebase + b * SUP
        pltpu.async_copy(gidx_hbm.at[pl.ds(eb, SUP)], gref[w], sem_i[w])
        pltpu.async_copy(s_hbm.at[pl.ds(eb, SUP)], sref[w], sem_i[w])
        pltpu.async_copy(dl_hbm.at[pl.ds(dlbase + b * SUP, SUP)],
                         dref[w], sem_i[w])

    def wait_idx(w):
        pltpu.make_async_copy(gidx_hbm.at[pl.ds(0, SUP)], gref[w],
                              sem_i[w]).wait()
        pltpu.make_async_copy(s_hbm.at[pl.ds(0, SUP)], sref[w],
                              sem_i[w]).wait()
        pltpu.make_async_copy(dl_hbm.at[pl.ds(0, SUP)], dref[w],
                              sem_i[w]).wait()

    def scale(ssup, off, rref):
        def sb(i, _):
            for k in range(16):
                e = i * 16 + k
                sk = plsc.load_gather(ssup, [jnp.full((16,), off + e,
                                                      jnp.int32)])
                for p in range(4):
                    sl = pl.ds(p * 16, 16)
                    rref[e, sl] = rref[e, sl] * sk
            return 0
        lax.fori_loop(0, CH // 16, sb, 0)

    def super_body(sup, w):
        gsup, ssup, dsup = gref[w], sref[w], dref[w]
        wait_idx(w)
        pltpu.async_copy(y_hbm.at[gsup.at[pl.ds(0, CH)]], rows0, sem_g0)
        rows = [rows0, rows1]
        for j in range(CPB):
            t = j % 2
            rref = rows[t]
            if j + 1 < CPB:
                if j >= 1:
                    pltpu.make_async_copy(rows[1 - t], acc_sh.at[pl.ds(0, CH)],
                                          sem_s).wait()
                pltpu.async_copy(y_hbm.at[gsup.at[pl.ds((j + 1) * CH, CH)]],
                                 rows[1 - t], sem_g[1 - t])
            pltpu.make_async_copy(y_hbm.at[pl.ds(0, CH)], rref,
                                  sem_g[t]).wait()
            scale(ssup, j * CH, rref)
            pltpu.async_copy(rref, acc_sh.at[dsup.at[pl.ds(j * CH, CH)]],
                             sem_s, add=True)
        pltpu.make_async_copy(rows0, acc_sh.at[pl.ds(0, CH)], sem_s).wait()
        pltpu.make_async_copy(rows1, acc_sh.at[pl.ds(0, CH)], sem_s).wait()

    load_idx(0, 0)

    def sup_loop(m, _):
        sup = 2 * m
        load_idx(sup + 1, 1)
        super_body(sup, 0)

        @pl.when(sup + 2 < NSUP)
        def _pre():
            load_idx(sup + 2, 0)
        super_body(sup + 1, 1)
        return 0
    lax.fori_loop(0, NSUP // 2, sup_loop, 0)

    plsc.subcore_barrier()

    @pl.when(sid < 15)
    def _out_main():
        pltpu.sync_copy(acc_sh.at[pl.ds(r0, 1568)],
                        out_hbm.at[pl.ds(nbase + r0, 1568)])

    @pl.when(sid == 15)
    def _out_tail():
        pltpu.sync_copy(acc_sh.at[pl.ds(23520, 1480)],
                        out_hbm.at[pl.ds(nbase + 23520, 1480)])


def kernel(x_user, W_user, b_user, item_emb, W_rel, W_root, bias, edge_index, edge_type):
    h_user = _user_matmul(x_user, W_user, b_user)
    x = jnp.concatenate([h_user, item_emb], axis=0)

    src = edge_index[0]
    dst = edge_index[1]
    pad = E_PAD - E_K
    src_p = jnp.concatenate([src, jnp.zeros((pad,), jnp.int32)])
    dst_p = jnp.concatenate([dst, jnp.full((pad,), PAD_DST, jnp.int32)])
    t_p = jnp.concatenate([edge_type, jnp.zeros((pad,), jnp.int32)])

    inv = _sc_prep1(dst_p, t_p)
    gidx, s_e, dl = _sc_prep2(src_p, dst_p, t_p, inv)

    for l in range(L_K):
        root, y = _layer_matmul(x, W_root[l], W_rel[l], bias[l], relu=(l > 0))
        x = _sc_layer(root, y.reshape(R_K * N_K, H_K), gidx, s_e, dl)
    return x


# speedup vs baseline: 1.4552x; 1.4552x over previous
import functools
import jax
import jax.numpy as jnp
from jax import lax
from jax.experimental import pallas as pl
from jax.experimental.pallas import tpu as pltpu
from jax.experimental.pallas import tpu_sc as plsc

N_K = 50000
H_K = 64
R_K = 4
L_K = 3
E_K = 800000
E_PAD = 802816
PAD_DST = 50001

N_HALF = 25000
TRASH_ROW = 25000
ACC_ROWS = 25008
CH = 112
CPB = 4
EB = CH * H_K * 4

CT_HALF = 102400
CT_TRASH = 100096
INV_SZ = 2 * CT_HALF

EPT_P1 = 50176
EPW_P2 = 25088

_MESH = plsc.VectorSubcoreMesh(core_axis_name="c", subcore_axis_name="s")
_SC_PARAMS = pltpu.CompilerParams(needs_layout_passes=False,
                                  use_tc_tiling_on_sc=False)


def _user_mm_body(x_ref, w_ref, b_ref, o_ref):
    o_ref[...] = jnp.dot(x_ref[...], w_ref[...],
                         preferred_element_type=jnp.float32) + b_ref[...]


def _user_matmul(x_user, W_user, b_user):
    M, K = x_user.shape
    H = W_user.shape[1]
    BM = 1000
    return pl.pallas_call(
        _user_mm_body,
        grid=(M // BM,),
        in_specs=[
            pl.BlockSpec((BM, K), lambda i: (i, 0)),
            pl.BlockSpec((K, H), lambda i: (0, 0)),
            pl.BlockSpec((1, H), lambda i: (0, 0)),
        ],
        out_specs=pl.BlockSpec((BM, H), lambda i: (i, 0)),
        out_shape=jax.ShapeDtypeStruct((M, H), jnp.float32),
    )(x_user, W_user, b_user.reshape(1, H))


def _layer_mm_body(relu, x_ref, wroot_ref, wrel_ref, bias_ref, root_ref, y_ref):
    x = x_ref[...]
    if relu:
        x = jnp.maximum(x, 0.0)
    root_ref[...] = jnp.dot(x, wroot_ref[...],
                            preferred_element_type=jnp.float32) + bias_ref[...]
    for r in range(R_K):
        y_ref[r] = jnp.dot(x, wrel_ref[r],
                           preferred_element_type=jnp.float32)


def _layer_matmul(x, W_root_l, W_rel_l, bias_l, relu):
    N, H = x.shape
    BM = 1000
    return pl.pallas_call(
        functools.partial(_layer_mm_body, relu),
        grid=(N // BM,),
        in_specs=[
            pl.BlockSpec((BM, H), lambda i: (i, 0)),
            pl.BlockSpec((H, H), lambda i: (0, 0)),
            pl.BlockSpec((R_K, H, H), lambda i: (0, 0, 0)),
            pl.BlockSpec((1, H), lambda i: (0, 0)),
        ],
        out_specs=[
            pl.BlockSpec((BM, H), lambda i: (i, 0)),
            pl.BlockSpec((R_K, BM, H), lambda i: (0, i, 0)),
        ],
        out_shape=[
            jax.ShapeDtypeStruct((N, H), jnp.float32),
            jax.ShapeDtypeStruct((R_K, N, H), jnp.float32),
        ],
    )(x, W_root_l, W_rel_l, bias_l.reshape(1, H))


@functools.partial(
    pl.kernel, mesh=_MESH, compiler_params=_SC_PARAMS,
    out_type=jax.ShapeDtypeStruct((INV_SZ,), jnp.float32),
    scratch_types=[
        pltpu.VMEM_SHARED((CT_HALF,), jnp.float32),
        pltpu.VMEM((6400,), jnp.float32),
        pltpu.VMEM((512,), jnp.float32),
        pltpu.VMEM((512,), jnp.int32),
        pltpu.VMEM((512,), jnp.int32),
        pltpu.VMEM((512,), jnp.int32),
        pltpu.VMEM((512,), jnp.int32),
        pltpu.VMEM((512,), jnp.int32),
        pltpu.VMEM((512,), jnp.int32),
        pltpu.SemaphoreType.DMA,
        pltpu.SemaphoreType.DMA,
        pltpu.SemaphoreType.DMA,
        pltpu.SemaphoreType.DMA,
    ],
)
def _sc_prep1(dst_hbm, t_hbm, inv_hbm, counts_sh, zbuf, ones_b,
              db0, db1, tb0, tb1, cb0, cb1, sem_i0, sem_i1, sem_s0, sem_s1):
    cid = lax.axis_index("c")
    sid = lax.axis_index("s")
    dref = [db0, db1]
    tref = [tb0, tb1]
    cref = [cb0, cb1]
    sem_i = [sem_i0, sem_i1]
    NCH = EPT_P1 // 512

    def zero_body(i, _):
        zbuf[pl.ds(i * 16, 16)] = jnp.zeros((16,), jnp.float32)
        return 0
    lax.fori_loop(0, 400, zero_body, 0)

    def ones_body(i, _):
        ones_b[pl.ds(i * 16, 16)] = jnp.ones((16,), jnp.float32)
        return 0
    lax.fori_loop(0, 32, ones_body, 0)

    pltpu.sync_copy(zbuf.at[pl.ds(0, 6400)],
                    counts_sh.at[pl.ds(sid * 6400, 6400)])
    plsc.subcore_barrier()

    nbase = cid * N_HALF
    ebase = sid * EPT_P1

    def load(ch, w):
        pltpu.async_copy(dst_hbm.at[pl.ds(ebase + ch * 512, 512)],
                         dref[w], sem_i[w])
        pltpu.async_copy(t_hbm.at[pl.ds(ebase + ch * 512, 512)],
                         tref[w], sem_i[w])

    def wait_load(w):
        pltpu.make_async_copy(dst_hbm.at[pl.ds(0, 512)], dref[w],
                              sem_i[w]).wait()
        pltpu.make_async_copy(t_hbm.at[pl.ds(0, 512)], tref[w],
                              sem_i[w]).wait()

    sem_s = [sem_s0, sem_s1]

    def wait_scat(w):
        pltpu.make_async_copy(ones_b, counts_sh.at[pl.ds(0, 512)],
                              sem_s[w]).wait()

    def process(ch, w, first):
        wait_load(w)
        if not first:
            wait_scat(w)
        cb = cref[w]

        def cbody(k, _c):
            sl = pl.ds(k * 16, 16)
            d16 = dref[w][sl]
            t16 = tref[w][sl]
            loc = d16 - nbase
            ok = (loc >= 0) & (loc < N_HALF)
            cb[sl] = jnp.where(ok, loc * R_K + t16, CT_TRASH)
            return 0
        lax.fori_loop(0, 32, cbody, 0)
        pltpu.async_copy(ones_b, counts_sh.at[cb], sem_s[w], add=True)

    load(0, 0)
    load(1, 1)
    process(0, 0, True)
    load(2, 0)
    process(1, 1, True)
    load(3, 1)
    process(2, 0, False)

    def ch_loop(m, _):
        ch = 2 * m + 3
        load(ch + 1, 0)
        process(ch, 1, False)

        @pl.when(ch + 2 < NCH)
        def _l1():
            load(ch + 2, 1)
        process(ch + 1, 0, False)
        return 0
    lax.fori_loop(0, (NCH - 4) // 2, ch_loop, 0)

    process(NCH - 1, 1, False)
    wait_scat(0)
    wait_scat(1)

    plsc.subcore_barrier()
    pltpu.sync_copy(counts_sh.at[pl.ds(sid * 6400, 6400)],
                    zbuf.at[pl.ds(0, 6400)])

    def inv_body(i, _):
        v = zbuf[pl.ds(i * 16, 16)]
        zbuf[pl.ds(i * 16, 16)] = 1.0 / jnp.maximum(v, 1.0)
        return 0
    lax.fori_loop(0, 400, inv_body, 0)
    pltpu.sync_copy(zbuf.at[pl.ds(0, 6400)],
                    inv_hbm.at[pl.ds(cid * CT_HALF + sid * 6400, 6400)])


@functools.partial(
    pl.kernel, mesh=_MESH, compiler_params=_SC_PARAMS,
    out_type=[
        jax.ShapeDtypeStruct((E_PAD,), jnp.int32),
        jax.ShapeDtypeStruct((E_PAD,), jnp.float32),
        jax.ShapeDtypeStruct((2 * E_PAD,), jnp.int32),
    ],
    scratch_types=[
        pltpu.VMEM((512,), jnp.int32),
        pltpu.VMEM((512,), jnp.int32),
        pltpu.VMEM((512,), jnp.int32),
        pltpu.VMEM((512,), jnp.int32),
        pltpu.VMEM((512,), jnp.int32),
        pltpu.VMEM((512,), jnp.int32),
        pltpu.VMEM((512,), jnp.int32),
        pltpu.VMEM((512,), jnp.int32),
        pltpu.VMEM((512,), jnp.int32),
        pltpu.VMEM((512,), jnp.int32),
        pltpu.VMEM((512,), jnp.float32),
        pltpu.VMEM((512,), jnp.float32),
        pltpu.VMEM((512,), jnp.int32),
        pltpu.VMEM((512,), jnp.int32),
        pltpu.VMEM((512,), jnp.int32),
        pltpu.VMEM((512,), jnp.int32),
        pltpu.SemaphoreType.DMA,
        pltpu.SemaphoreType.DMA,
        pltpu.SemaphoreType.DMA,
        pltpu.SemaphoreType.DMA,
        pltpu.SemaphoreType.DMA,
    ],
)
def _sc_prep2(src_hbm, dst_hbm, t_hbm, inv_hbm, gidx_hbm, s_hbm, dl_hbm,
              bsrc0, bsrc1, bdst0, bdst1, bt0, bt1, gf0, gf1, cf0, cf1,
              sf0, sf1, d0f0, d0f1, d1f0, d1f1,
              sem_l0, sem_l1, sem_g, sem_st0, sem_st1):
    cid = lax.axis_index("c")
    sid = lax.axis_index("s")
    wid = cid * 16 + sid
    wbase = wid * EPW_P2
    NCW = EPW_P2 // 512

    bsrc = [bsrc0, bsrc1]
    bdst = [bdst0, bdst1]
    bt = [bt0, bt1]
    gf = [gf0, gf1]
    cf = [cf0, cf1]
    sf = [sf0, sf1]
    d0f = [d0f0, d0f1]
    d1f = [d1f0, d1f1]
    sem_st = [sem_st0, sem_st1]
    sem_l = [sem_l0, sem_l1]

    def load(ch, w):
        base = wbase + ch * 512
        pltpu.async_copy(src_hbm.at[pl.ds(base, 512)], bsrc[w], sem_l[w])
        pltpu.async_copy(dst_hbm.at[pl.ds(base, 512)], bdst[w], sem_l[w])
        pltpu.async_copy(t_hbm.at[pl.ds(base, 512)], bt[w], sem_l[w])

    def wait_load(w):
        for _ in range(3):
            pltpu.make_async_copy(src_hbm.at[pl.ds(0, 512)], bsrc[w],
                                  sem_l[w]).wait()

    def wait_st(w):
        for _ in range(3):
            pltpu.make_async_copy(gf[w], gidx_hbm.at[pl.ds(0, 512)],
                                  sem_st[w]).wait()
        pltpu.make_async_copy(sf[w], s_hbm.at[pl.ds(0, 512)],
                              sem_st[w]).wait()

    def process(ch, w, first):
        base = wbase + ch * 512
        wait_load(w)
        if not first:
            wait_st(w)

        def cbody(i, _c):
            sl = pl.ds(i * 16, 16)
            s16 = bsrc[w][sl]
            d16 = bdst[w][sl]
            t16 = bt[w][sl]
            gf[w][sl] = t16 * N_K + s16
            upper = d16 >= N_HALF
            loc = d16 - jnp.where(upper, N_HALF, 0)
            cf[w][sl] = jnp.where(upper, CT_HALF, 0) + loc * R_K + t16
            d0f[w][sl] = jnp.where(d16 < N_HALF, d16, TRASH_ROW)
            loc1 = d16 - N_HALF
            ok1 = (loc1 >= 0) & (loc1 < N_HALF)
            d1f[w][sl] = jnp.where(ok1, loc1, TRASH_ROW)
            return 0
        lax.fori_loop(0, 32, cbody, 0)

        @pl.when(ch + 2 < NCW)
        def _prefetch():
            load(ch + 2, w)
        pltpu.async_copy(inv_hbm.at[cf[w]], sf[w], sem_g)
        pltpu.async_copy(gf[w], gidx_hbm.at[pl.ds(base, 512)], sem_st[w])
        pltpu.async_copy(d0f[w], dl_hbm.at[pl.ds(base, 512)], sem_st[w])
        pltpu.async_copy(d1f[w], dl_hbm.at[pl.ds(E_PAD + base, 512)],
                         sem_st[w])
        pltpu.make_async_copy(inv_hbm.at[pl.ds(0, 512)], sf[w], sem_g).wait()
        pltpu.async_copy(sf[w], s_hbm.at[pl.ds(base, 512)], sem_st[w])

    load(0, 0)
    load(1, 1)
    process(0, 0, True)
    process(1, 1, True)

    def ch_loop(m, _):
        process(2 * m + 2, 0, False)
        process(2 * m + 3, 1, False)
        return 0
    lax.fori_loop(0, (NCW - 3) // 2, ch_loop, 0)

    process(NCW - 1, 0, False)
    wait_st(1)
    wait_st(0)


@functools.partial(
    pl.kernel, mesh=_MESH, compiler_params=_SC_PARAMS,
    out_type=jax.ShapeDtypeStruct((N_K, H_K), jnp.float32),
    scratch_types=[
        pltpu.VMEM_SHARED((ACC_ROWS, H_K), jnp.float32),
        pltpu.VMEM((CPB * CH,), jnp.int32),
        pltpu.VMEM((CPB * CH,), jnp.int32),
        pltpu.VMEM((CPB * CH,), jnp.float32),
        pltpu.VMEM((CPB * CH,), jnp.float32),
        pltpu.VMEM((CPB * CH,), jnp.int32),
        pltpu.VMEM((CPB * CH,), jnp.int32),
        pltpu.VMEM((CH, H_K), jnp.float32),
        pltpu.VMEM((CH, H_K), jnp.float32),
        pltpu.SemaphoreType.DMA,
        pltpu.SemaphoreType.DMA,
        pltpu.SemaphoreType.DMA,
        pltpu.SemaphoreType.DMA,
        pltpu.SemaphoreType.DMA,
    ],
)
def _sc_layer(root_hbm, y_hbm, gidx_hbm, s_hbm, dl_hbm, out_hbm,
              acc_sh, gsup0, gsup1, ssup0, ssup1, dsup0, dsup1,
              rows0, rows1, sem_g0, sem_g1, sem_s, sem_i0, sem_i1):
    cid = lax.axis_index("c")
    sid = lax.axis_index("s")
    nbase = cid * N_HALF
    r0 = sid * 1568
    SUP = CPB * CH
    NSUP = EPT_P1 // SUP

    @pl.when(sid < 15)
    def _init_main():
        pltpu.sync_copy(root_hbm.at[pl.ds(nbase + r0, 1568)],
                        acc_sh.at[pl.ds(r0, 1568)])

    @pl.when(sid == 15)
    def _init_tail():
        pltpu.sync_copy(root_hbm.at[pl.ds(nbase + 23520, 1480)],
                        acc_sh.at[pl.ds(23520, 1480)])

    plsc.subcore_barrier()

    ebase = sid * EPT_P1
    dlbase = cid * E_PAD + ebase
    gref = [gsup0, gsup1]
    sref = [ssup0, ssup1]
    dref = [dsup0, dsup1]
    sem_i = [sem_i0, sem_i1]
    sem_g = [sem_g0, sem_g1]

    def load_idx(b, w):
        eb = ebase + b * SUP
        pltpu.async_copy(gidx_hbm.at[pl.ds(eb, SUP)], gref[w], sem_i[w])
        pltpu.async_copy(s_hbm.at[pl.ds(eb, SUP)], sref[w], sem_i[w])
        pltpu.async_copy(dl_hbm.at[pl.ds(dlbase + b * SUP, SUP)],
                         dref[w], sem_i[w])

    def wait_idx(w):
        pltpu.make_async_copy(gidx_hbm.at[pl.ds(0, SUP)], gref[w],
                              sem_i[w]).wait()
        pltpu.make_async_copy(s_hbm.at[pl.ds(0, SUP)], sref[w],
                              sem_i[w]).wait()
        pltpu.make_async_copy(dl_hbm.at[pl.ds(0, SUP)], dref[w],
                              sem_i[w]).wait()

    def scale(ssup, off, rref):
        def sb(i, _):
            for k in range(4):
                e = i * 4 + k
                sk = plsc.load_gather(ssup, [jnp.full((16,), off + e,
                                                      jnp.int32)])
                for p in range(4):
                    sl = pl.ds(p * 16, 16)
                    rref[e, sl] = rref[e, sl] * sk
            return 0
        lax.fori_loop(0, CH // 4, sb, 0)

    def super_body(sup, w):
        gsup, ssup, dsup = gref[w], sref[w], dref[w]
        wait_idx(w)
        pltpu.async_copy(y_hbm.at[gsup.at[pl.ds(0, CH)]], rows0, sem_g0)
        rows = [rows0, rows1]
        for j in range(CPB):
            t = j % 2
            rref = rows[t]
            if j + 1 < CPB:
                if j >= 1:
                    pltpu.make_async_copy(rows[1 - t], acc_sh.at[pl.ds(0, CH)],
                                          sem_s).wait()
                pltpu.async_copy(y_hbm.at[gsup.at[pl.ds((j + 1) * CH, CH)]],
                                 rows[1 - t], sem_g[1 - t])
            pltpu.make_async_copy(y_hbm.at[pl.ds(0, CH)], rref,
                                  sem_g[t]).wait()
            scale(ssup, j * CH, rref)
            pltpu.async_copy(rref, acc_sh.at[dsup.at[pl.ds(j * CH, CH)]],
                             sem_s, add=True)
        pltpu.make_async_copy(rows0, acc_sh.at[pl.ds(0, CH)], sem_s).wait()
        pltpu.make_async_copy(rows1, acc_sh.at[pl.ds(0, CH)], sem_s).wait()

    load_idx(0, 0)

    def sup_loop(m, _):
        sup = 2 * m
        load_idx(sup + 1, 1)
        super_body(sup, 0)

        @pl.when(sup + 2 < NSUP)
        def _pre():
            load_idx(sup + 2, 0)
        super_body(sup + 1, 1)
        return 0
    lax.fori_loop(0, NSUP // 2, sup_loop, 0)

    plsc.subcore_barrier()

    @pl.when(sid < 15)
    def _out_main():
        pltpu.sync_copy(acc_sh.at[pl.ds(r0, 1568)],
                        out_hbm.at[pl.ds(nbase + r0, 1568)])

    @pl.when(sid == 15)
    def _out_tail():
        pltpu.sync_copy(acc_sh.at[pl.ds(23520, 1480)],
                        out_hbm.at[pl.ds(nbase + 23520, 1480)])


def kernel(x_user, W_user, b_user, item_emb, W_rel, W_root, bias, edge_index, edge_type):
    h_user = _user_matmul(x_user, W_user, b_user)
    x = jnp.concatenate([h_user, item_emb], axis=0)

    src = edge_index[0]
    dst = edge_index[1]
    pad = E_PAD - E_K
    src_p = jnp.concatenate([src, jnp.zeros((pad,), jnp.int32)])
    dst_p = jnp.concatenate([dst, jnp.full((pad,), PAD_DST, jnp.int32)])
    t_p = jnp.concatenate([edge_type, jnp.zeros((pad,), jnp.int32)])

    inv = _sc_prep1(dst_p, t_p)
    gidx, s_e, dl = _sc_prep2(src_p, dst_p, t_p, inv)

    for l in range(L_K):
        root, y = _layer_matmul(x, W_root[l], W_rel[l], bias[l], relu=(l > 0))
        x = _sc_layer(root, y.reshape(R_K * N_K, H_K), gidx, s_e, dl)
    return x
